# Initial kernel scaffold; baseline (speedup 1.0000x reference)
import jax
import jax.numpy as jnp
from jax.experimental import pallas as pl


def kernel(x, champ_table, item_table, trait_table):
    raise NotImplementedError("write your pallas kernel here")


# speedup vs baseline: 11.5044x; 11.5044x over previous
import functools

import jax
import jax.numpy as jnp
import numpy as np
from jax.experimental import pallas as pl

B, S = 4096, 50
NUM_CHAMP, NUM_ITEM, NUM_TRAIT = 60, 60, 27
D_CHAMP, D_ITEM, D_TRAIT = 30, 10, 8
STATS = 31

_SEGS = (
    (0, NUM_CHAMP),
    (3, NUM_ITEM),
    (4, NUM_ITEM),
    (5, NUM_ITEM),
    (6, NUM_TRAIT),
    (7, NUM_TRAIT),
    (8, NUM_TRAIT),
    (9, NUM_TRAIT),
    (10, NUM_TRAIT),
    (11, NUM_TRAIT),
    (12, NUM_TRAIT),
    (1, 4),
    (2, 15),
)
K_TOTAL = sum(w for _, w in _SEGS)
D_EMB = D_CHAMP + 3 * D_ITEM + 7 * D_TRAIT
D_OUT = D_EMB + 4 + 15 + STATS

_ROWS = 2048


def _build_table(champ_table, item_table, trait_table):
    w = jnp.zeros((K_TOTAL, D_EMB + 19), jnp.float32)
    r, c = 0, 0
    w = w.at[r:r + NUM_CHAMP, c:c + D_CHAMP].set(champ_table)
    r += NUM_CHAMP
    c += D_CHAMP
    for _ in range(3):
        w = w.at[r:r + NUM_ITEM, c:c + D_ITEM].set(item_table)
        r += NUM_ITEM
        c += D_ITEM
    for _ in range(7):
        w = w.at[r:r + NUM_TRAIT, c:c + D_TRAIT].set(trait_table)
        r += NUM_TRAIT
        c += D_TRAIT
    w = w.at[r:r + 4, c:c + 4].set(jnp.eye(4, dtype=jnp.float32))
    r += 4
    c += 4
    w = w.at[r:r + 15, c:c + 15].set(jnp.eye(15, dtype=jnp.float32))
    return w


def _body(x_ref, w_ref, o_ref):
    xb = x_ref[:, :]
    pieces = []
    for col, width in _SEGS:
        ids = xb[:, col:col + 1].astype(jnp.int32)
        iota = jax.lax.broadcasted_iota(jnp.int32, (_ROWS, width), 1)
        pieces.append((iota == ids).astype(jnp.float32))
    onehot = jnp.concatenate(pieces, axis=1)
    emb = jnp.dot(onehot, w_ref[:, :], preferred_element_type=jnp.float32)
    o_ref[:, :] = jnp.concatenate([emb, xb[:, 13:]], axis=1)


@jax.jit
def kernel(x, champ_table, item_table, trait_table):
    w = _build_table(champ_table, item_table, trait_table)
    n = B * S
    xf = x.reshape(n, 13 + STATS)
    out = pl.pallas_call(
        _body,
        grid=(n // _ROWS,),
        in_specs=[
            pl.BlockSpec((_ROWS, 13 + STATS), lambda i: (i, 0)),
            pl.BlockSpec((K_TOTAL, D_EMB + 19), lambda i: (0, 0)),
        ],
        out_specs=pl.BlockSpec((_ROWS, D_OUT), lambda i: (i, 0)),
        out_shape=jax.ShapeDtypeStruct((n, D_OUT), jnp.float32),
    )(xf, w)
    return out.reshape(B, S, D_OUT)


# speedup vs baseline: 18.7012x; 1.6256x over previous
import jax
import jax.numpy as jnp
import numpy as np
from jax.experimental import pallas as pl

B, S = 4096, 50
NUM_CHAMP, NUM_ITEM, NUM_TRAIT = 60, 60, 27
D_CHAMP, D_ITEM, D_TRAIT = 30, 10, 8
STATS = 31

_SEGS = (
    (0, NUM_CHAMP),
    (3, NUM_ITEM), (4, NUM_ITEM), (5, NUM_ITEM),
    (6, NUM_TRAIT), (7, NUM_TRAIT), (8, NUM_TRAIT), (9, NUM_TRAIT),
    (10, NUM_TRAIT), (11, NUM_TRAIT), (12, NUM_TRAIT),
    (1, 4),
    (2, 15),
)
K_OH = sum(w for _, w in _SEGS)
K_ALL = K_OH + STATS
D_EMB = D_CHAMP + 3 * D_ITEM + 7 * D_TRAIT
D_OUT = D_EMB + 4 + 15 + STATS
D_IN = 13 + STATS

_ROWS = 2048

_sel = np.zeros((D_IN, K_ALL), np.float32)
_loc = np.zeros((1, K_ALL), np.float32)
_j = 0
for _col, _w in _SEGS:
    _sel[_col, _j:_j + _w] = 1.0
    _loc[0, _j:_j + _w] = np.arange(_w, dtype=np.float32)
    _j += _w
for _c in range(STATS):
    _sel[13 + _c, K_OH + _c] = 1.0


def _build_table(champ_table, item_table, trait_table):
    w = jnp.zeros((K_ALL, D_OUT), jnp.float32)
    r, c = 0, 0
    w = w.at[r:r + NUM_CHAMP, c:c + D_CHAMP].set(champ_table)
    r += NUM_CHAMP
    c += D_CHAMP
    for _ in range(3):
        w = w.at[r:r + NUM_ITEM, c:c + D_ITEM].set(item_table)
        r += NUM_ITEM
        c += D_ITEM
    for _ in range(7):
        w = w.at[r:r + NUM_TRAIT, c:c + D_TRAIT].set(trait_table)
        r += NUM_TRAIT
        c += D_TRAIT
    w = w.at[r:r + 4, c:c + 4].set(jnp.eye(4, dtype=jnp.float32))
    r += 4
    c += 4
    w = w.at[r:r + 15, c:c + 15].set(jnp.eye(15, dtype=jnp.float32))
    r += 15
    c += 15
    w = w.at[r:r + STATS, c:c + STATS].set(jnp.eye(STATS, dtype=jnp.float32))
    return w


def _body(x_ref, s_ref, l_ref, w_ref, o_ref):
    xb = x_ref[:, :]
    f = jnp.dot(xb, s_ref[:, :], preferred_element_type=jnp.float32)
    col = jax.lax.broadcasted_iota(jnp.int32, (_ROWS, K_ALL), 1)
    onehot = jnp.where(col < K_OH, (f == l_ref[:, :]).astype(jnp.float32), f)
    o_ref[:, :] = jnp.dot(onehot, w_ref[:, :], preferred_element_type=jnp.float32)


@jax.jit
def kernel(x, champ_table, item_table, trait_table):
    w = _build_table(champ_table, item_table, trait_table)
    sel = jnp.asarray(_sel)
    loc = jnp.asarray(_loc)
    n = B * S
    xf = x.reshape(n, D_IN)
    out = pl.pallas_call(
        _body,
        grid=(n // _ROWS,),
        in_specs=[
            pl.BlockSpec((_ROWS, D_IN), lambda i: (i, 0)),
            pl.BlockSpec((D_IN, K_ALL), lambda i: (0, 0)),
            pl.BlockSpec((1, K_ALL), lambda i: (0, 0)),
            pl.BlockSpec((K_ALL, D_OUT), lambda i: (0, 0)),
        ],
        out_specs=pl.BlockSpec((_ROWS, D_OUT), lambda i: (i, 0)),
        out_shape=jax.ShapeDtypeStruct((n, D_OUT), jnp.float32),
    )(xf, sel, loc, w)
    return out.reshape(B, S, D_OUT)


# speedup vs baseline: 28.7020x; 1.5348x over previous
import jax
import jax.numpy as jnp
import numpy as np
from jax.experimental import pallas as pl

B, S = 4096, 50
NUM_CHAMP, NUM_ITEM, NUM_TRAIT = 60, 60, 27
D_CHAMP, D_ITEM, D_TRAIT = 30, 10, 8
STATS = 31

_SEGS = (
    (0, NUM_CHAMP),
    (3, NUM_ITEM), (4, NUM_ITEM), (5, NUM_ITEM),
    (6, NUM_TRAIT), (7, NUM_TRAIT), (8, NUM_TRAIT), (9, NUM_TRAIT),
    (10, NUM_TRAIT), (11, NUM_TRAIT), (12, NUM_TRAIT),
    (1, 4),
    (2, 15),
)
K_OH = sum(w for _, w in _SEGS)
D_EMB = D_CHAMP + 3 * D_ITEM + 7 * D_TRAIT
D_MM = D_EMB + 4 + 15
D_OUT = D_MM + STATS
D_IN = 13 + STATS

_RB = 64
_R = _RB * S

_sel = np.zeros((D_IN, K_OH), np.float32)
_loc = np.zeros((1, K_OH), np.float32)
_j = 0
for _col, _w in _SEGS:
    _sel[_col, _j:_j + _w] = 1.0
    _loc[0, _j:_j + _w] = np.arange(_w, dtype=np.float32)
    _j += _w


def _build_table(champ_table, item_table, trait_table):
    w = jnp.zeros((K_OH, D_MM), jnp.float32)
    r, c = 0, 0
    w = w.at[r:r + NUM_CHAMP, c:c + D_CHAMP].set(champ_table)
    r += NUM_CHAMP
    c += D_CHAMP
    for _ in range(3):
        w = w.at[r:r + NUM_ITEM, c:c + D_ITEM].set(item_table)
        r += NUM_ITEM
        c += D_ITEM
    for _ in range(7):
        w = w.at[r:r + NUM_TRAIT, c:c + D_TRAIT].set(trait_table)
        r += NUM_TRAIT
        c += D_TRAIT
    w = w.at[r:r + 4, c:c + 4].set(jnp.eye(4, dtype=jnp.float32))
    r += 4
    c += 4
    w = w.at[r:r + 15, c:c + 15].set(jnp.eye(15, dtype=jnp.float32))
    return w.astype(jnp.bfloat16)


def _body(x_ref, s_ref, l_ref, w_ref, o_ref):
    xb = jnp.reshape(x_ref[...], (_R, D_IN))
    f = jnp.dot(xb.astype(jnp.bfloat16), s_ref[...],
                preferred_element_type=jnp.float32)
    onehot = (f == l_ref[...]).astype(jnp.bfloat16)
    emb = jnp.dot(onehot, w_ref[...], preferred_element_type=jnp.float32)
    out2 = jnp.concatenate([emb, xb[:, 13:]], axis=1)
    o_ref[...] = jnp.reshape(out2, (_RB, S, D_OUT))


@jax.jit
def kernel(x, champ_table, item_table, trait_table):
    w = _build_table(champ_table, item_table, trait_table)
    sel = jnp.asarray(_sel, dtype=jnp.bfloat16)
    loc = jnp.asarray(_loc)
    out = pl.pallas_call(
        _body,
        grid=(B // _RB,),
        in_specs=[
            pl.BlockSpec((_RB, S, D_IN), lambda i: (i, 0, 0)),
            pl.BlockSpec((D_IN, K_OH), lambda i: (0, 0)),
            pl.BlockSpec((1, K_OH), lambda i: (0, 0)),
            pl.BlockSpec((K_OH, D_MM), lambda i: (0, 0)),
        ],
        out_specs=pl.BlockSpec((_RB, S, D_OUT), lambda i: (i, 0, 0)),
        out_shape=jax.ShapeDtypeStruct((B, S, D_OUT), jnp.float32),
    )(x, sel, loc, w)
    return out


# speedup vs baseline: 50.6461x; 1.7645x over previous
import jax
import jax.numpy as jnp
import numpy as np
from jax.experimental import pallas as pl

B, S = 4096, 50
NUM_CHAMP, NUM_ITEM, NUM_TRAIT = 60, 60, 27
D_CHAMP, D_ITEM, D_TRAIT = 30, 10, 8
STATS = 31

_SEGS = (
    (0, NUM_CHAMP),
    (3, NUM_ITEM), (4, NUM_ITEM), (5, NUM_ITEM),
    (6, NUM_TRAIT), (7, NUM_TRAIT), (8, NUM_TRAIT), (9, NUM_TRAIT),
    (10, NUM_TRAIT), (11, NUM_TRAIT), (12, NUM_TRAIT),
    (1, 4),
    (2, 15),
)
K_OH = sum(w for _, w in _SEGS)
D_EMB = D_CHAMP + 3 * D_ITEM + 7 * D_TRAIT
D_MM = D_EMB + 4 + 15
D_OUT = D_MM + STATS
D_IN = 13 + STATS

_NB = 256

_sel = np.zeros((K_OH, D_IN), np.float32)
_loc = np.zeros((K_OH, 1), np.float32)
_j = 0
for _col, _w in _SEGS:
    _sel[_j:_j + _w, _col] = 1.0
    _loc[_j:_j + _w, 0] = np.arange(_w, dtype=np.float32)
    _j += _w


def _build_table(champ_table, item_table, trait_table):
    w = jnp.zeros((K_OH, D_MM), jnp.float32)
    r, c = 0, 0
    w = w.at[r:r + NUM_CHAMP, c:c + D_CHAMP].set(champ_table)
    r += NUM_CHAMP
    c += D_CHAMP
    for _ in range(3):
        w = w.at[r:r + NUM_ITEM, c:c + D_ITEM].set(item_table)
        r += NUM_ITEM
        c += D_ITEM
    for _ in range(7):
        w = w.at[r:r + NUM_TRAIT, c:c + D_TRAIT].set(trait_table)
        r += NUM_TRAIT
        c += D_TRAIT
    w = w.at[r:r + 4, c:c + 4].set(jnp.eye(4, dtype=jnp.float32))
    r += 4
    c += 4
    w = w.at[r:r + 15, c:c + 15].set(jnp.eye(15, dtype=jnp.float32))
    return w.T.astype(jnp.bfloat16)


def _body(x_ref, s_ref, l_ref, w_ref, o_ref):
    s_mat = s_ref[...]
    l_col = l_ref[...]
    w_mat = w_ref[...]
    for s in range(S):
        xs = x_ref[s]
        f = jnp.dot(s_mat, xs.astype(jnp.bfloat16),
                    preferred_element_type=jnp.float32)
        onehot = (f == l_col).astype(jnp.bfloat16)
        emb = jnp.dot(w_mat, onehot,
                      preferred_element_type=jnp.float32)
        o_ref[s] = jnp.concatenate([emb, xs[13:, :]], axis=0)


@jax.jit
def kernel(x, champ_table, item_table, trait_table):
    w = _build_table(champ_table, item_table, trait_table)
    sel = jnp.asarray(_sel, dtype=jnp.bfloat16)
    loc = jnp.asarray(_loc)
    xt = jnp.transpose(x, (1, 2, 0))
    out_t = pl.pallas_call(
        _body,
        grid=(B // _NB,),
        in_specs=[
            pl.BlockSpec((S, D_IN, _NB), lambda i: (0, 0, i)),
            pl.BlockSpec((K_OH, D_IN), lambda i: (0, 0)),
            pl.BlockSpec((K_OH, 1), lambda i: (0, 0)),
            pl.BlockSpec((D_MM, K_OH), lambda i: (0, 0)),
        ],
        out_specs=pl.BlockSpec((S, D_OUT, _NB), lambda i: (0, 0, i)),
        out_shape=jax.ShapeDtypeStruct((S, D_OUT, B), jnp.float32),
    )(xt, sel, loc, w)
    return jnp.transpose(out_t, (2, 0, 1))


# speedup vs baseline: 87.0200x; 1.7182x over previous
import jax
import jax.numpy as jnp
import numpy as np
from jax.experimental import pallas as pl

B, S = 4096, 50
NUM_CHAMP, NUM_ITEM, NUM_TRAIT = 60, 60, 27
D_CHAMP, D_ITEM, D_TRAIT = 30, 10, 8
STATS = 31

_SEGS = (
    (0, NUM_CHAMP),
    (3, NUM_ITEM), (4, NUM_ITEM), (5, NUM_ITEM),
    (6, NUM_TRAIT), (7, NUM_TRAIT), (8, NUM_TRAIT), (9, NUM_TRAIT),
    (10, NUM_TRAIT), (11, NUM_TRAIT), (12, NUM_TRAIT),
    (1, 4),
    (2, 15),
)
K_OH = sum(w for _, w in _SEGS)
D_EMB = D_CHAMP + 3 * D_ITEM + 7 * D_TRAIT
D_MM = D_EMB + 4 + 15
D_OUT = D_MM + STATS
D_IN = 13 + STATS

_NB = 2048

_sel = np.zeros((K_OH, D_IN), np.float32)
_loc = np.zeros((K_OH, 1), np.float32)
_j = 0
for _col, _w in _SEGS:
    _sel[_j:_j + _w, _col] = 1.0
    _loc[_j:_j + _w, 0] = np.arange(_w, dtype=np.float32)
    _j += _w


def _build_table(champ_table, item_table, trait_table):
    w = jnp.zeros((K_OH, D_MM), jnp.float32)
    r, c = 0, 0
    w = w.at[r:r + NUM_CHAMP, c:c + D_CHAMP].set(champ_table)
    r += NUM_CHAMP
    c += D_CHAMP
    for _ in range(3):
        w = w.at[r:r + NUM_ITEM, c:c + D_ITEM].set(item_table)
        r += NUM_ITEM
        c += D_ITEM
    for _ in range(7):
        w = w.at[r:r + NUM_TRAIT, c:c + D_TRAIT].set(trait_table)
        r += NUM_TRAIT
        c += D_TRAIT
    w = w.at[r:r + 4, c:c + 4].set(jnp.eye(4, dtype=jnp.float32))
    r += 4
    c += 4
    w = w.at[r:r + 15, c:c + 15].set(jnp.eye(15, dtype=jnp.float32))
    return w.T.astype(jnp.bfloat16)


def _body(x_ref, s_ref, l_ref, w_ref, o_ref):
    xs = x_ref[0]
    f = jnp.dot(s_ref[...], xs.astype(jnp.bfloat16),
                preferred_element_type=jnp.float32)
    onehot = (f == l_ref[...]).astype(jnp.bfloat16)
    emb = jnp.dot(w_ref[...], onehot,
                  preferred_element_type=jnp.float32)
    o_ref[0] = jnp.concatenate([emb, xs[13:, :]], axis=0)


@jax.jit
def kernel(x, champ_table, item_table, trait_table):
    w = _build_table(champ_table, item_table, trait_table)
    sel = jnp.asarray(_sel, dtype=jnp.bfloat16)
    loc = jnp.asarray(_loc)
    xt = jnp.transpose(x, (1, 2, 0))
    out_t = pl.pallas_call(
        _body,
        grid=(S, B // _NB),
        in_specs=[
            pl.BlockSpec((1, D_IN, _NB), lambda s, i: (s, 0, i)),
            pl.BlockSpec((K_OH, D_IN), lambda s, i: (0, 0)),
            pl.BlockSpec((K_OH, 1), lambda s, i: (0, 0)),
            pl.BlockSpec((D_MM, K_OH), lambda s, i: (0, 0)),
        ],
        out_specs=pl.BlockSpec((1, D_OUT, _NB), lambda s, i: (s, 0, i)),
        out_shape=jax.ShapeDtypeStruct((S, D_OUT, B), jnp.float32),
    )(xt, sel, loc, w)
    return jnp.transpose(out_t, (2, 0, 1))


# speedup vs baseline: 105.8618x; 1.2165x over previous
import jax
import jax.numpy as jnp
import numpy as np
from jax.experimental import pallas as pl

B, S = 4096, 50
NUM_CHAMP, NUM_ITEM, NUM_TRAIT = 60, 60, 27
D_CHAMP, D_ITEM, D_TRAIT = 30, 10, 8
STATS = 31

_SEGS = (
    (0, NUM_CHAMP),
    (3, NUM_ITEM), (4, NUM_ITEM), (5, NUM_ITEM),
    (6, NUM_TRAIT), (7, NUM_TRAIT), (8, NUM_TRAIT), (9, NUM_TRAIT),
    (10, NUM_TRAIT), (11, NUM_TRAIT), (12, NUM_TRAIT),
    (1, 4),
    (2, 15),
)
K_OH = sum(w for _, w in _SEGS)
D_EMB = D_CHAMP + 3 * D_ITEM + 7 * D_TRAIT
D_MM = D_EMB + 4 + 15
D_OUT = D_MM + STATS
D_IN = 13 + STATS

_NB = 4096

_sel = np.zeros((K_OH, D_IN), np.float32)
_loc = np.zeros((K_OH, 1), np.float32)
_j = 0
for _col, _w in _SEGS:
    _sel[_j:_j + _w, _col] = 1.0
    _loc[_j:_j + _w, 0] = np.arange(_w, dtype=np.float32)
    _j += _w


def _build_table(champ_table, item_table, trait_table):
    w = jnp.zeros((K_OH, D_MM), jnp.float32)
    r, c = 0, 0
    w = w.at[r:r + NUM_CHAMP, c:c + D_CHAMP].set(champ_table)
    r += NUM_CHAMP
    c += D_CHAMP
    for _ in range(3):
        w = w.at[r:r + NUM_ITEM, c:c + D_ITEM].set(item_table)
        r += NUM_ITEM
        c += D_ITEM
    for _ in range(7):
        w = w.at[r:r + NUM_TRAIT, c:c + D_TRAIT].set(trait_table)
        r += NUM_TRAIT
        c += D_TRAIT
    w = w.at[r:r + 4, c:c + 4].set(jnp.eye(4, dtype=jnp.float32))
    r += 4
    c += 4
    w = w.at[r:r + 15, c:c + 15].set(jnp.eye(15, dtype=jnp.float32))
    return w.T.astype(jnp.bfloat16)


def _body(x_ref, s_ref, l_ref, w_ref, o_ref):
    xs = x_ref[0]
    f = jnp.dot(s_ref[...], xs.astype(jnp.bfloat16),
                preferred_element_type=jnp.float32)
    onehot = (f == l_ref[...]).astype(jnp.bfloat16)
    emb = jnp.dot(w_ref[...], onehot,
                  preferred_element_type=jnp.float32)
    o_ref[0] = jnp.concatenate([emb, xs[13:, :]], axis=0)


@jax.jit
def kernel(x, champ_table, item_table, trait_table):
    w = _build_table(champ_table, item_table, trait_table)
    sel = jnp.asarray(_sel, dtype=jnp.bfloat16)
    loc = jnp.asarray(_loc)
    xt = jnp.transpose(x, (1, 2, 0))
    out_t = pl.pallas_call(
        _body,
        grid=(S, B // _NB),
        in_specs=[
            pl.BlockSpec((1, D_IN, _NB), lambda s, i: (s, 0, i)),
            pl.BlockSpec((K_OH, D_IN), lambda s, i: (0, 0)),
            pl.BlockSpec((K_OH, 1), lambda s, i: (0, 0)),
            pl.BlockSpec((D_MM, K_OH), lambda s, i: (0, 0)),
        ],
        out_specs=pl.BlockSpec((1, D_OUT, _NB), lambda s, i: (s, 0, i)),
        out_shape=jax.ShapeDtypeStruct((S, D_OUT, B), jnp.float32),
    )(xt, sel, loc, w)
    return jnp.transpose(out_t, (2, 0, 1))


# speedup vs baseline: 114.3638x; 1.0803x over previous
import jax
import jax.numpy as jnp
import numpy as np
from jax.experimental import pallas as pl

B, S = 4096, 50
NUM_CHAMP, NUM_ITEM, NUM_TRAIT = 60, 60, 27
D_CHAMP, D_ITEM, D_TRAIT = 30, 10, 8
STATS = 31

_SEGS = (
    (0, NUM_CHAMP),
    (3, NUM_ITEM), (4, NUM_ITEM), (5, NUM_ITEM),
    (6, NUM_TRAIT), (7, NUM_TRAIT), (8, NUM_TRAIT), (9, NUM_TRAIT),
    (10, NUM_TRAIT), (11, NUM_TRAIT), (12, NUM_TRAIT),
    (1, 4),
    (2, 15),
)
K_OH = sum(w for _, w in _SEGS)
D_EMB = D_CHAMP + 3 * D_ITEM + 7 * D_TRAIT
D_MM = D_EMB + 4 + 15
D_OUT = D_MM + STATS
D_IN = 13 + STATS

_NB = 4096

_sel = np.zeros((K_OH, D_IN), np.float32)
_loc = np.zeros((K_OH, 1), np.float32)
_j = 0
for _col, _w in _SEGS:
    _sel[_j:_j + _w, _col] = 1.0
    _loc[_j:_j + _w, 0] = np.arange(_w, dtype=np.float32)
    _j += _w


def _build_table(champ_table, item_table, trait_table):
    w = jnp.zeros((K_OH, D_MM), jnp.float32)
    r, c = 0, 0
    w = w.at[r:r + NUM_CHAMP, c:c + D_CHAMP].set(champ_table)
    r += NUM_CHAMP
    c += D_CHAMP
    for _ in range(3):
        w = w.at[r:r + NUM_ITEM, c:c + D_ITEM].set(item_table)
        r += NUM_ITEM
        c += D_ITEM
    for _ in range(7):
        w = w.at[r:r + NUM_TRAIT, c:c + D_TRAIT].set(trait_table)
        r += NUM_TRAIT
        c += D_TRAIT
    w = w.at[r:r + 4, c:c + 4].set(jnp.eye(4, dtype=jnp.float32))
    r += 4
    c += 4
    w = w.at[r:r + 15, c:c + 15].set(jnp.eye(15, dtype=jnp.float32))
    return w.T.astype(jnp.bfloat16)


_SB = 2


def _body(x_ref, s_ref, l_ref, w_ref, o_ref):
    for s in range(_SB):
        xs = x_ref[s]
        f = jnp.dot(s_ref[...], xs.astype(jnp.bfloat16),
                    preferred_element_type=jnp.float32)
        onehot = (f == l_ref[...]).astype(jnp.bfloat16)
        emb = jnp.dot(w_ref[...], onehot,
                      preferred_element_type=jnp.float32)
        o_ref[s] = jnp.concatenate([emb, xs[13:, :]], axis=0)


@jax.jit
def kernel(x, champ_table, item_table, trait_table):
    w = _build_table(champ_table, item_table, trait_table)
    sel = jnp.asarray(_sel, dtype=jnp.bfloat16)
    loc = jnp.asarray(_loc)
    xt = jnp.transpose(x, (1, 2, 0))
    out_t = pl.pallas_call(
        _body,
        grid=(S // _SB, B // _NB),
        in_specs=[
            pl.BlockSpec((_SB, D_IN, _NB), lambda s, i: (s, 0, i)),
            pl.BlockSpec((K_OH, D_IN), lambda s, i: (0, 0)),
            pl.BlockSpec((K_OH, 1), lambda s, i: (0, 0)),
            pl.BlockSpec((D_MM, K_OH), lambda s, i: (0, 0)),
        ],
        out_specs=pl.BlockSpec((_SB, D_OUT, _NB), lambda s, i: (s, 0, i)),
        out_shape=jax.ShapeDtypeStruct((S, D_OUT, B), jnp.float32),
    )(xt, sel, loc, w)
    return jnp.transpose(out_t, (2, 0, 1))
